# baseline (speedup 1.0000x reference)
import functools
import jax
import jax.numpy as jnp
from jax import lax
from jax.experimental import pallas as pl
from jax.experimental.pallas import tpu as pltpu
from jax.experimental.pallas import tpu_sc as plsc

N = 100000
IN_CH = 128
B = 32
CH = 384
NCHUNK = 8
WROWS = CH * NCHUNK

mesh = plsc.VectorSubcoreMesh(core_axis_name="c", subcore_axis_name="s")


@functools.partial(
    pl.kernel, mesh=mesh,
    out_type=jax.ShapeDtypeStruct((N,), jnp.float32),
    scratch_types=[
        pltpu.VMEM((CH, IN_CH), jnp.float32),
        pltpu.VMEM((CH,), jnp.int32),
        pltpu.VMEM((CH,), jnp.float32),
        pltpu.VMEM((B, IN_CH), jnp.float32),
    ],
)
def _sc_e_pass(x_hbm, idx_hbm, q_hbm, e_hbm, xv, iv, ev, qv):
    c = lax.axis_index("c")
    s = lax.axis_index("s")
    w = s * 2 + c
    pltpu.sync_copy(q_hbm, qv)
    lane = lax.iota(jnp.int32, 16)

    def chunk_body(ci, carry):
        base = w * WROWS + ci * CH
        pltpu.sync_copy(x_hbm.at[pl.ds(base, CH)], xv)
        pltpu.sync_copy(idx_hbm.at[pl.ds(base, CH)], iv)

        def grp_body(gi, carry2):
            iv16 = iv[pl.ds(gi * 16, 16)]
            e16 = jnp.zeros((16,), jnp.float32)
            for j in range(16):
                b = iv16[j]
                r = gi * 16 + j
                acc = xv[r, pl.ds(0, 16)] * qv[b, pl.ds(0, 16)]
                for g in range(1, 8):
                    acc = acc + xv[r, pl.ds(16 * g, 16)] * qv[b, pl.ds(16 * g, 16)]
                for sh in (1, 2, 4, 8):
                    rot = lax.gather(
                        acc, ((lane + sh) % 16)[:, None],
                        lax.GatherDimensionNumbers(
                            offset_dims=(), collapsed_slice_dims=(0,),
                            start_index_map=(0,)),
                        (1,), mode=lax.GatherScatterMode.PROMISE_IN_BOUNDS)
                    acc = acc + rot
                e16 = jnp.where(lane == j, acc, e16)
            ev[pl.ds(gi * 16, 16)] = e16
            return carry2

        lax.fori_loop(0, CH // 16, grp_body, 0)
        pltpu.sync_copy(ev, e_hbm.at[pl.ds(base, CH)])
        return carry

    lax.fori_loop(0, NCHUNK, chunk_body, 0)


def kernel(x, batch_index, W_ih, W_hh, b_ih, b_hh):
    q = x[:B, :]
    e = None
    for _ in range(3):
        e = _sc_e_pass(x, batch_index, q)
        q = q + e[:B, None] * 1e-9
    out = jnp.zeros((B, 2 * IN_CH), jnp.float32) + e[:B, None]
    return out


# speedup vs baseline: 6.8390x; 6.8390x over previous
import jax
import jax.numpy as jnp
from jax import lax
from jax.experimental import pallas as pl
from jax.experimental.pallas import tpu as pltpu

N = 100000
IN_CH = 128
OUT_CH = 2 * IN_CH
B = 32
STEPS = 3
DBLK = 20000
NB = N // DBLK
NRESC = 2
NSB = 2

_NEG = -1e30


def _body(x_hbm, idx_ref, wi_ref, wh_ref, b_ref, out_ref,
          h_ref, c_ref, qs_ref, m_ref, s_ref, r_ref,
          res_ref, sbuf_ref, sem):
    t = pl.program_id(0)
    b = pl.program_id(1)

    def res_dma(c):
        return pltpu.make_async_copy(
            x_hbm.at[pl.ds(c * DBLK, DBLK), :],
            res_ref.at[pl.ds(c * DBLK, DBLK), :],
            sem.at[c])

    def sbuf_dma(c):
        return pltpu.make_async_copy(
            x_hbm.at[pl.ds(c * DBLK, DBLK), :],
            sbuf_ref.at[(c - NRESC) % NSB],
            sem.at[c])

    @pl.when((t == 0) & (b == 0))
    def _issue_t0():
        for c in range(NRESC):
            res_dma(c).start()
        for c in range(NRESC, NRESC + NSB):
            sbuf_dma(c).start()

    @pl.when((t > 0) & (b == 0))
    def _issue_t():
        for c in range(NRESC, NRESC + NSB):
            sbuf_dma(c).start()

    @pl.when(b == 0)
    def _head():
        @pl.when(t == 0)
        def _init():
            h_ref[...] = jnp.zeros((B, IN_CH), jnp.float32)
            c_ref[...] = jnp.zeros((B, IN_CH), jnp.float32)
            qs_ref[...] = jnp.zeros((B, OUT_CH), jnp.float32)

        @pl.when(t > 0)
        def _finalize_prev():
            r = jnp.where(m_ref[...] > _NEG,
                          r_ref[...] / (s_ref[...] + 1e-16), 0.0)
            qs_ref[:, :IN_CH] = h_ref[...]
            qs_ref[:, IN_CH:] = r

        qs = qs_ref[...]
        gates = (jnp.dot(qs, wi_ref[...], preferred_element_type=jnp.float32)
                 + jnp.dot(h_ref[...], wh_ref[...], preferred_element_type=jnp.float32)
                 + b_ref[...])
        ig = jax.nn.sigmoid(gates[:, :IN_CH])
        fg = jax.nn.sigmoid(gates[:, IN_CH:2 * IN_CH])
        gg = jnp.tanh(gates[:, 2 * IN_CH:3 * IN_CH])
        og = jax.nn.sigmoid(gates[:, 3 * IN_CH:])
        c_new = fg * c_ref[...] + ig * gg
        h_new = og * jnp.tanh(c_new)
        c_ref[...] = c_new
        h_ref[...] = h_new
        m_ref[...] = jnp.full((B, 1), _NEG, jnp.float32)
        s_ref[...] = jnp.zeros((B, 1), jnp.float32)
        r_ref[...] = jnp.zeros((B, IN_CH), jnp.float32)

    def process(xb, chunk):
        idxc = idx_ref[chunk]
        q = h_ref[...]

        seg = lax.broadcasted_iota(jnp.int32, (B, DBLK), 0)
        mask = idxc == seg

        et = lax.dot_general(q, xb, (((1,), (1,)), ((), ())),
                             preferred_element_type=jnp.float32)
        em = jnp.where(mask, et, _NEG)

        m_old = m_ref[...]
        m_blk = jnp.max(em, axis=1, keepdims=True)
        m_new = jnp.maximum(m_old, m_blk)
        scale = jnp.exp(m_old - m_new)

        p = jnp.exp(em - m_new)

        m_ref[...] = m_new
        s_ref[...] = s_ref[...] * scale + jnp.sum(p, axis=1, keepdims=True)
        r_ref[...] = r_ref[...] * scale + lax.dot_general(
            p, xb, (((1,), (0,)), ((), ())), preferred_element_type=jnp.float32)

    @pl.when(b < NRESC)
    def _from_resident():
        @pl.when(t == 0)
        def _wait_fill():
            res_dma(b).wait()
        process(res_ref[pl.ds(b * DBLK, DBLK), :], b)

    @pl.when(b >= NRESC)
    def _from_stream():
        sbuf_dma(b).wait()
        process(sbuf_ref[(b - NRESC) % NSB], b)

    @pl.when((b >= NRESC) & (b + NSB < NB))
    def _issue_trailing():
        sbuf_dma(b + NSB).start()

    @pl.when((t == STEPS - 1) & (b == NB - 1))
    def _emit():
        out_ref[:, :IN_CH] = h_ref[...]
        out_ref[:, IN_CH:] = jnp.where(m_ref[...] > _NEG,
                                       r_ref[...] / (s_ref[...] + 1e-16), 0.0)


def kernel(x, batch_index, W_ih, W_hh, b_ih, b_hh):
    wi = W_ih.T
    wh = W_hh.T
    bias = (b_ih + b_hh).reshape(1, 4 * IN_CH)
    idx3 = batch_index.reshape(NB, 1, DBLK)

    return pl.pallas_call(
        _body,
        grid=(STEPS, NB),
        in_specs=[
            pl.BlockSpec(memory_space=pl.ANY),
            pl.BlockSpec((NB, 1, DBLK), lambda t, b: (0, 0, 0)),
            pl.BlockSpec((OUT_CH, 4 * IN_CH), lambda t, b: (0, 0)),
            pl.BlockSpec((IN_CH, 4 * IN_CH), lambda t, b: (0, 0)),
            pl.BlockSpec((1, 4 * IN_CH), lambda t, b: (0, 0)),
        ],
        out_specs=pl.BlockSpec((B, OUT_CH), lambda t, b: (0, 0)),
        out_shape=jax.ShapeDtypeStruct((B, OUT_CH), jnp.float32),
        scratch_shapes=[
            pltpu.VMEM((B, IN_CH), jnp.float32),
            pltpu.VMEM((B, IN_CH), jnp.float32),
            pltpu.VMEM((B, OUT_CH), jnp.float32),
            pltpu.VMEM((B, 1), jnp.float32),
            pltpu.VMEM((B, 1), jnp.float32),
            pltpu.VMEM((B, IN_CH), jnp.float32),
            pltpu.VMEM((NRESC * DBLK, IN_CH), jnp.float32),
            pltpu.VMEM((NSB, DBLK, IN_CH), jnp.float32),
            pltpu.SemaphoreType.DMA((NB,)),
        ],
        compiler_params=pltpu.CompilerParams(
            dimension_semantics=("arbitrary", "arbitrary"),
        ),
    )(x, idx3, wi, wh, bias)
